# baseline (device time: 13354 ns/iter reference)
import jax
import jax.numpy as jnp
from jax import lax
from jax.experimental import pallas as pl
from jax.experimental.pallas import tpu as pltpu

K = 8


def kernel(x, pi):
    m, h, w = x.shape
    half = h // 2
    ch = half // K

    def body(
        pi_ref,
        x_hbm,
        out_hbm,
        stage,
        send_buf,
        recv_buf,
        out_stage,
        copy_sem,
        out_sems,
        ysend,
        yrecv,
        xsend,
        xrecv,
    ):
        my_x = lax.axis_index("x")
        my_y = lax.axis_index("y")
        my_z = lax.axis_index("z")
        dst_y = pi_ref[my_y]
        other_x = 1 - my_x
        base = my_x * half
        other_base = other_x * half

        local = pltpu.make_async_copy(
            x_hbm.at[0, pl.ds(base, half), :], stage, copy_sem
        )
        local.start()

        barrier_sem = pltpu.get_barrier_semaphore()
        pl.semaphore_signal(
            barrier_sem,
            inc=1,
            device_id=(my_x, dst_y, my_z),
            device_id_type=pl.DeviceIdType.MESH,
        )
        pl.semaphore_signal(
            barrier_sem,
            inc=1,
            device_id=(other_x, my_y, my_z),
            device_id_type=pl.DeviceIdType.MESH,
        )
        pl.semaphore_wait(barrier_sem, 2)
        local.wait()

        y_rdmas = []
        for k in range(K):
            rl = pl.ds(k * ch, ch)
            send_buf[rl, :] = stage[rl, :].astype(jnp.bfloat16)
            rd = pltpu.make_async_remote_copy(
                src_ref=send_buf.at[rl],
                dst_ref=recv_buf.at[pl.ds(base + k * ch, ch)],
                send_sem=ysend.at[k],
                recv_sem=yrecv.at[k],
                device_id=(my_x, dst_y, my_z),
                device_id_type=pl.DeviceIdType.MESH,
            )
            rd.start()
            y_rdmas.append(rd)

        x_rdmas = []
        out_dmas = []
        for k in range(K):
            y_rdmas[k].wait_recv()
            r = pl.ds(base + k * ch, ch)
            rd = pltpu.make_async_remote_copy(
                src_ref=recv_buf.at[r],
                dst_ref=recv_buf.at[r],
                send_sem=xsend.at[k],
                recv_sem=xrecv.at[k],
                device_id=(other_x, my_y, my_z),
                device_id_type=pl.DeviceIdType.MESH,
            )
            rd.start()
            x_rdmas.append(rd)
            out_stage[r, :] = recv_buf[r, :].astype(jnp.float32)
            od = pltpu.make_async_copy(
                out_stage.at[r], out_hbm.at[0, r, :], out_sems.at[k]
            )
            od.start()
            out_dmas.append(od)

        for k in range(K):
            x_rdmas[k].wait_recv()
            r = pl.ds(other_base + k * ch, ch)
            out_stage[r, :] = recv_buf[r, :].astype(jnp.float32)
            od = pltpu.make_async_copy(
                out_stage.at[r], out_hbm.at[0, r, :], out_sems.at[K + k]
            )
            od.start()
            out_dmas.append(od)

        for od in out_dmas:
            od.wait()
        for k in range(K):
            y_rdmas[k].wait_send()
            x_rdmas[k].wait_send()

    return pl.pallas_call(
        body,
        out_shape=jax.ShapeDtypeStruct((m, h, w), jnp.float32),
        in_specs=[
            pl.BlockSpec(memory_space=pltpu.SMEM),
            pl.BlockSpec(memory_space=pltpu.HBM),
        ],
        out_specs=pl.BlockSpec(memory_space=pltpu.HBM),
        scratch_shapes=[
            pltpu.VMEM((half, w), jnp.float32),
            pltpu.VMEM((half, w), jnp.bfloat16),
            pltpu.VMEM((h, w), jnp.bfloat16),
            pltpu.VMEM((h, w), jnp.float32),
            pltpu.SemaphoreType.DMA,
            pltpu.SemaphoreType.DMA((2 * K,)),
            pltpu.SemaphoreType.DMA((K,)),
            pltpu.SemaphoreType.DMA((K,)),
            pltpu.SemaphoreType.DMA((K,)),
            pltpu.SemaphoreType.DMA((K,)),
        ],
        compiler_params=pltpu.CompilerParams(collective_id=0),
    )(pi, x)


# device time: 13280 ns/iter; 1.0056x vs baseline; 1.0056x over previous
import jax
import jax.numpy as jnp
from jax import lax
from jax.experimental import pallas as pl
from jax.experimental.pallas import tpu as pltpu

K = 8


def kernel(x, pi):
    m, h, w = x.shape
    ch = h // K

    def body(
        pi_ref,
        x_hbm,
        out_hbm,
        stage,
        send_buf,
        recv_buf,
        out_stage,
        copy_sem,
        out_sems,
        ysend,
        yrecv,
    ):
        my_x = lax.axis_index("x")
        my_y = lax.axis_index("y")
        my_z = lax.axis_index("z")
        dst_y = pi_ref[my_y]

        local = pltpu.make_async_copy(x_hbm.at[0], stage, copy_sem)
        local.start()
        local.wait()
        send_buf[...] = stage[...].astype(jnp.bfloat16)

        barrier_sem = pltpu.get_barrier_semaphore()
        pl.semaphore_signal(
            barrier_sem,
            inc=1,
            device_id=(my_x, dst_y, my_z),
            device_id_type=pl.DeviceIdType.MESH,
        )
        pl.semaphore_wait(barrier_sem, 1)

        y_rdmas = []
        for k in range(K):
            r = pl.ds(k * ch, ch)
            rd = pltpu.make_async_remote_copy(
                src_ref=send_buf.at[r],
                dst_ref=recv_buf.at[r],
                send_sem=ysend.at[k],
                recv_sem=yrecv.at[k],
                device_id=(my_x, dst_y, my_z),
                device_id_type=pl.DeviceIdType.MESH,
            )
            rd.start()
            y_rdmas.append(rd)

        out_dmas = []
        for k in range(K):
            y_rdmas[k].wait_recv()
            r = pl.ds(k * ch, ch)
            out_stage[r, :] = recv_buf[r, :].astype(jnp.float32)
            od = pltpu.make_async_copy(
                out_stage.at[r], out_hbm.at[0, r, :], out_sems.at[k]
            )
            od.start()
            out_dmas.append(od)

        for k in range(K):
            out_dmas[k].wait()
            y_rdmas[k].wait_send()

    return pl.pallas_call(
        body,
        out_shape=jax.ShapeDtypeStruct((m, h, w), jnp.float32),
        in_specs=[
            pl.BlockSpec(memory_space=pltpu.SMEM),
            pl.BlockSpec(memory_space=pltpu.HBM),
        ],
        out_specs=pl.BlockSpec(memory_space=pltpu.HBM),
        scratch_shapes=[
            pltpu.VMEM((h, w), jnp.float32),
            pltpu.VMEM((h, w), jnp.bfloat16),
            pltpu.VMEM((h, w), jnp.bfloat16),
            pltpu.VMEM((h, w), jnp.float32),
            pltpu.SemaphoreType.DMA,
            pltpu.SemaphoreType.DMA((K,)),
            pltpu.SemaphoreType.DMA((K,)),
            pltpu.SemaphoreType.DMA((K,)),
        ],
        compiler_params=pltpu.CompilerParams(collective_id=0),
    )(pi, x)


# device time: 12898 ns/iter; 1.0354x vs baseline; 1.0296x over previous
import jax
import jax.numpy as jnp
from jax import lax
from jax.experimental import pallas as pl
from jax.experimental.pallas import tpu as pltpu

K = 4


def kernel(x, pi):
    m, h, w = x.shape
    ch = h // K

    def body(
        pi_ref,
        x_hbm,
        out_ref,
        stage,
        send_buf,
        recv_buf,
        copy_sem,
        ysend,
        yrecv,
    ):
        my_x = lax.axis_index("x")
        my_y = lax.axis_index("y")
        my_z = lax.axis_index("z")
        dst_y = pi_ref[my_y]

        local = pltpu.make_async_copy(x_hbm.at[0], stage, copy_sem)
        local.start()

        barrier_sem = pltpu.get_barrier_semaphore()
        pl.semaphore_signal(
            barrier_sem,
            inc=1,
            device_id=(my_x, dst_y, my_z),
            device_id_type=pl.DeviceIdType.MESH,
        )
        pl.semaphore_wait(barrier_sem, 1)
        local.wait()

        y_rdmas = []
        for k in range(K):
            r = pl.ds(k * ch, ch)
            send_buf[r, :] = stage[r, :].astype(jnp.bfloat16)
            rd = pltpu.make_async_remote_copy(
                src_ref=send_buf.at[r],
                dst_ref=recv_buf.at[r],
                send_sem=ysend.at[k],
                recv_sem=yrecv.at[k],
                device_id=(my_x, dst_y, my_z),
                device_id_type=pl.DeviceIdType.MESH,
            )
            rd.start()
            y_rdmas.append(rd)

        for k in range(K):
            y_rdmas[k].wait_recv()
            r = pl.ds(k * ch, ch)
            out_ref[0, r, :] = recv_buf[r, :].astype(jnp.float32)

        for k in range(K):
            y_rdmas[k].wait_send()

    return pl.pallas_call(
        body,
        out_shape=jax.ShapeDtypeStruct((m, h, w), jnp.float32),
        in_specs=[
            pl.BlockSpec(memory_space=pltpu.SMEM),
            pl.BlockSpec(memory_space=pltpu.HBM),
        ],
        out_specs=pl.BlockSpec(memory_space=pltpu.VMEM),
        scratch_shapes=[
            pltpu.VMEM((h, w), jnp.float32),
            pltpu.VMEM((h, w), jnp.bfloat16),
            pltpu.VMEM((h, w), jnp.bfloat16),
            pltpu.SemaphoreType.DMA,
            pltpu.SemaphoreType.DMA((K,)),
            pltpu.SemaphoreType.DMA((K,)),
        ],
        compiler_params=pltpu.CompilerParams(collective_id=0),
    )(pi, x)
